# baseline (speedup 1.0000x reference)
import functools

import jax
import jax.numpy as jnp
from jax import lax
from jax.experimental import pallas as pl
from jax.experimental.pallas import tpu as pltpu
from jax.experimental.pallas import tpu_sc as plsc

B, Q, C, R, V = 64, 300, 92, 100, 117
NC, NS, L = 2, 16, 16
NW = NC * NS
BPW = B // NW
NG = (R + L - 1) // L
NCLS = 81

_mesh = plsc.VectorSubcoreMesh(
    core_axis_name="c", subcore_axis_name="s", num_cores=NC, num_subcores=NS)


def _splat(x, dtype=jnp.int32):
    return jnp.zeros((L,), dtype) + x


def _iota():
    return lax.iota(jnp.int32, L)


def _batch_scratch():
    return [
        pltpu.VMEM((R * 2,), jnp.int32),
        pltpu.VMEM((Q * C,), jnp.float32),
        pltpu.VMEM((Q * 4,), jnp.float32),
        pltpu.VMEM((2 * R,), jnp.int32),
        pltpu.VMEM((2 * R * 4,), jnp.float32),
    ]


@functools.partial(
    pl.kernel,
    out_type=(
        jax.ShapeDtypeStruct((B * 2 * R,), jnp.int32),
        jax.ShapeDtypeStruct((B * 2 * R * 4,), jnp.float32),
        jax.ShapeDtypeStruct((B * R,), jnp.float32),
    ),
    mesh=_mesh,
    compiler_params=pltpu.CompilerParams(
        needs_layout_passes=False, use_tc_tiling_on_sc=True),
    scratch_types=[pltpu.VMEM((B * 2,), jnp.float32),
                   pltpu.VMEM((BPW * R,), jnp.float32)] + _batch_scratch()
    + _batch_scratch()
    + [pltpu.SemaphoreType.DMA, pltpu.SemaphoreType.DMA,
       pltpu.SemaphoreType.DMA],
)
def _scpart(logits_hbm, boxes_hbm, pairs_hbm, ts_hbm,
            l_hbm, b_hbm, sc_hbm, ts_v, scores_v, *rest):
    per_batch = (rest[0:5], rest[5:10])
    sems = rest[10:12]
    sem_out = rest[12]
    wid = lax.axis_index("s") * NC + lax.axis_index("c")
    b0 = wid * BPW

    copies = []
    for j in range(BPW):
        pairs_v, logits_v, boxes_v = per_batch[j][:3]
        bj = b0 + j
        copies.append([
            pltpu.async_copy(
                logits_hbm.at[pl.ds(bj * Q * C, Q * C)], logits_v, sems[j]),
            pltpu.async_copy(
                boxes_hbm.at[pl.ds(bj * Q * 4, Q * 4)], boxes_v, sems[j]),
            pltpu.async_copy(
                pairs_hbm.at[pl.ds(bj * R * 2, R * 2)], pairs_v, sems[j]),
        ])
    pltpu.sync_copy(ts_hbm, ts_v)
    out_copies = []

    for j in range(BPW):
        pairs_v, logits_v, boxes_v, l_buf, b_buf = per_batch[j]
        for c in copies[j]:
            c.wait()
        img_h = plsc.load_gather(ts_v, [_splat((b0 + j) * 2 + 0)])
        img_w = plsc.load_gather(ts_v, [_splat((b0 + j) * 2 + 1)])

        pis, masks, ois = [], [], []
        for g in range(NG):
            pi_raw = _iota() + g * L
            masks.append((pi_raw < R) if g == NG - 1 else None)
            pis.append(jnp.minimum(pi_raw, R - 1))
            ois.append(plsc.load_gather(pairs_v, [pis[g] * 2 + 1]))

        @plsc.parallel_loop(0, NG)
        def _boxes(g, pairs_v=pairs_v, boxes_v=boxes_v, l_buf=l_buf,
                   b_buf=b_buf, img_h=img_h, img_w=img_w):
            pi_raw = _iota() + g * L
            mask = pi_raw < R
            pi = jnp.minimum(pi_raw, R - 1)
            plsc.store_scatter(l_buf, [pi], _splat(0), mask=mask)
            for side in range(2):
                bi = plsc.load_gather(pairs_v, [pi * 2 + side])
                cx = plsc.load_gather(boxes_v, [bi * 4 + 0])
                cy = plsc.load_gather(boxes_v, [bi * 4 + 1])
                w = plsc.load_gather(boxes_v, [bi * 4 + 2])
                h = plsc.load_gather(boxes_v, [bi * 4 + 3])
                ob = (pi + side * R) * 4
                plsc.store_scatter(b_buf, [ob + 0],
                                   (cx - 0.5 * w) * img_w, mask=mask)
                plsc.store_scatter(b_buf, [ob + 1],
                                   (cy - 0.5 * h) * img_h, mask=mask)
                plsc.store_scatter(b_buf, [ob + 2],
                                   (cx + 0.5 * w) * img_w, mask=mask)
                plsc.store_scatter(b_buf, [ob + 3],
                                   (cy + 0.5 * h) * img_h, mask=mask)

        obase = [oi * C for oi in ois]
        init = (tuple(_splat(-jnp.inf, jnp.float32) for _ in range(NG)),
                tuple(_splat(0) for _ in range(NG)))

        @plsc.parallel_loop(0, NCLS - 1, unroll=4, carry=init)
        def maxstate(c, carry, obase=obase, logits_v=logits_v):
            ms, ams = carry
            ms, ams = list(ms), list(ams)
            cs = _splat(c)
            for g in range(NG):
                v = plsc.load_gather(logits_v, [obase[g] + cs])
                gt = v > ms[g]
                ms[g] = jnp.where(gt, v, ms[g])
                ams[g] = jnp.where(gt, cs, ams[g])
            return tuple(ms), tuple(ams)

        m80s, ams = maxstate
        malls = []
        for g in range(NG):
            v80 = plsc.load_gather(logits_v, [obase[g] + (NCLS - 1)])
            malls.append(jnp.maximum(m80s[g], v80))
        malls = tuple(malls)

        @plsc.parallel_loop(0, NCLS, unroll=3,
                            carry=tuple(_splat(0.0, jnp.float32)
                                        for _ in range(NG)))
        def ss(c, ss_c, obase=obase, malls=malls, logits_v=logits_v):
            ss_c = list(ss_c)
            cs = _splat(c)
            for g in range(NG):
                v = plsc.load_gather(logits_v, [obase[g] + cs])
                ss_c[g] = ss_c[g] + jnp.exp(v - malls[g])
            return tuple(ss_c)

        for g in range(NG):
            score = jnp.exp(m80s[g] - malls[g]) / ss[g]
            plsc.store_scatter(scores_v, [pis[g] + j * R], score,
                               mask=masks[g])
            plsc.store_scatter(l_buf, [pis[g] + R], ams[g], mask=masks[g])

        out_copies.extend([
            pltpu.async_copy(
                l_buf, l_hbm.at[pl.ds((b0 + j) * 2 * R, 2 * R)], sem_out),
            pltpu.async_copy(
                b_buf, b_hbm.at[pl.ds((b0 + j) * 2 * R * 4, 2 * R * 4)],
                sem_out),
        ])

    out_copies.append(pltpu.async_copy(
        scores_v, sc_hbm.at[pl.ds(b0 * R, BPW * R)], sem_out))
    for c in out_copies:
        c.wait()


def _vs_body(act_ref, sc_ref, vs_ref):
    vs_ref[...] = jax.nn.sigmoid(act_ref[...]) * sc_ref[...][:, :, None]


_vs_tc = pl.pallas_call(
    _vs_body,
    out_shape=jax.ShapeDtypeStruct((B, R, V), jnp.float32),
    grid=(B // 8,),
    in_specs=[
        pl.BlockSpec((8, R, V), lambda i: (i, 0, 0)),
        pl.BlockSpec((8, R), lambda i: (i, 0)),
    ],
    out_specs=pl.BlockSpec((8, R, V), lambda i: (i, 0, 0)),
)


def kernel(pred_logits, pred_boxes, pred_actions, pred_rel_pairs, target_sizes):
    l1, b1, sc1 = _scpart(
        pred_logits.reshape(-1), pred_boxes.reshape(-1),
        pred_rel_pairs.reshape(-1), target_sizes.reshape(-1))
    vs = _vs_tc(pred_actions, sc1.reshape(B, R))
    return (l1.reshape(B, 2 * R), b1.reshape(B, 2 * R, 4), vs)


# speedup vs baseline: 1.1060x; 1.1060x over previous
import functools

import jax
import jax.numpy as jnp
from jax import lax
from jax.experimental import pallas as pl
from jax.experimental.pallas import tpu as pltpu
from jax.experimental.pallas import tpu_sc as plsc

B, Q, C, R, V = 64, 300, 92, 100, 117
NC, NS, L = 2, 16, 16
NW = NC * NS
BPW = B // NW
NG = (R + L - 1) // L
NCLS = 81

_mesh = plsc.VectorSubcoreMesh(
    core_axis_name="c", subcore_axis_name="s", num_cores=NC, num_subcores=NS)


def _splat(x, dtype=jnp.int32):
    return jnp.zeros((L,), dtype) + x


def _iota():
    return lax.iota(jnp.int32, L)


def _batch_scratch():
    return [
        pltpu.VMEM((R * 2,), jnp.int32),
        pltpu.VMEM((Q, C), jnp.float32),
        pltpu.VMEM((Q * 4,), jnp.float32),
        pltpu.VMEM((2 * R,), jnp.int32),
        pltpu.VMEM((2 * R * 4,), jnp.float32),
    ]


@functools.partial(
    pl.kernel,
    out_type=(
        jax.ShapeDtypeStruct((B * 2 * R,), jnp.int32),
        jax.ShapeDtypeStruct((B * 2 * R * 4,), jnp.float32),
        jax.ShapeDtypeStruct((B * R,), jnp.float32),
    ),
    mesh=_mesh,
    compiler_params=pltpu.CompilerParams(
        needs_layout_passes=False, use_tc_tiling_on_sc=True),
    scratch_types=[pltpu.VMEM((B * 2,), jnp.float32),
                   pltpu.VMEM((BPW * R,), jnp.float32)] + _batch_scratch()
    + _batch_scratch()
    + [pltpu.SemaphoreType.DMA, pltpu.SemaphoreType.DMA,
       pltpu.SemaphoreType.DMA],
)
def _scpart(logits_hbm, boxes_hbm, pairs_hbm, ts_hbm,
            l_hbm, b_hbm, sc_hbm, ts_v, scores_v, *rest):
    per_batch = (rest[0:5], rest[5:10])
    sems = rest[10:12]
    sem_out = rest[12]
    wid = lax.axis_index("s") * NC + lax.axis_index("c")
    b0 = wid * BPW

    copies = []
    for j in range(BPW):
        pairs_v, logits_v, boxes_v = per_batch[j][:3]
        bj = b0 + j
        copies.append([
            pltpu.async_copy(logits_hbm.at[bj], logits_v, sems[j]),
            pltpu.async_copy(
                boxes_hbm.at[pl.ds(bj * Q * 4, Q * 4)], boxes_v, sems[j]),
            pltpu.async_copy(
                pairs_hbm.at[pl.ds(bj * R * 2, R * 2)], pairs_v, sems[j]),
        ])
    pltpu.sync_copy(ts_hbm, ts_v)
    out_copies = []

    for j in range(BPW):
        pairs_v, logits_v, boxes_v, l_buf, b_buf = per_batch[j]
        for c in copies[j]:
            c.wait()
        img_h = plsc.load_gather(ts_v, [_splat((b0 + j) * 2 + 0)])
        img_w = plsc.load_gather(ts_v, [_splat((b0 + j) * 2 + 1)])

        pis, masks, ois = [], [], []
        for g in range(NG):
            pi_raw = _iota() + g * L
            masks.append((pi_raw < R) if g == NG - 1 else None)
            pis.append(jnp.minimum(pi_raw, R - 1))
            ois.append(plsc.load_gather(pairs_v, [pis[g] * 2 + 1]))

        @plsc.parallel_loop(0, NG)
        def _boxes(g, pairs_v=pairs_v, boxes_v=boxes_v, l_buf=l_buf,
                   b_buf=b_buf, img_h=img_h, img_w=img_w):
            pi_raw = _iota() + g * L
            mask = pi_raw < R
            pi = jnp.minimum(pi_raw, R - 1)
            plsc.store_scatter(l_buf, [pi], _splat(0), mask=mask)
            for side in range(2):
                bi = plsc.load_gather(pairs_v, [pi * 2 + side])
                cx = plsc.load_gather(boxes_v, [bi * 4 + 0])
                cy = plsc.load_gather(boxes_v, [bi * 4 + 1])
                w = plsc.load_gather(boxes_v, [bi * 4 + 2])
                h = plsc.load_gather(boxes_v, [bi * 4 + 3])
                ob = (pi + side * R) * 4
                plsc.store_scatter(b_buf, [ob + 0],
                                   (cx - 0.5 * w) * img_w, mask=mask)
                plsc.store_scatter(b_buf, [ob + 1],
                                   (cy - 0.5 * h) * img_h, mask=mask)
                plsc.store_scatter(b_buf, [ob + 2],
                                   (cx + 0.5 * w) * img_w, mask=mask)
                plsc.store_scatter(b_buf, [ob + 3],
                                   (cy + 0.5 * h) * img_h, mask=mask)

        init = (tuple(_splat(-jnp.inf, jnp.float32) for _ in range(NG)),
                tuple(_splat(0) for _ in range(NG)))

        @plsc.parallel_loop(0, NCLS - 1, unroll=4, carry=init)
        def maxstate(c, carry, ois=ois, logits_v=logits_v):
            ms, ams = carry
            ms, ams = list(ms), list(ams)
            cs = _splat(c)
            for g in range(NG):
                v = plsc.load_gather(logits_v, [ois[g], cs])
                gt = v > ms[g]
                ms[g] = jnp.where(gt, v, ms[g])
                ams[g] = jnp.where(gt, cs, ams[g])
            return tuple(ms), tuple(ams)

        m80s, ams = maxstate
        malls = []
        for g in range(NG):
            v80 = plsc.load_gather(logits_v, [ois[g], _splat(NCLS - 1)])
            malls.append(jnp.maximum(m80s[g], v80))
        malls = tuple(malls)

        @plsc.parallel_loop(0, NCLS, unroll=3,
                            carry=tuple(_splat(0.0, jnp.float32)
                                        for _ in range(NG)))
        def ss(c, ss_c, ois=ois, malls=malls, logits_v=logits_v):
            ss_c = list(ss_c)
            cs = _splat(c)
            for g in range(NG):
                v = plsc.load_gather(logits_v, [ois[g], cs])
                ss_c[g] = ss_c[g] + jnp.exp(v - malls[g])
            return tuple(ss_c)

        for g in range(NG):
            score = jnp.exp(m80s[g] - malls[g]) / ss[g]
            plsc.store_scatter(scores_v, [pis[g] + j * R], score,
                               mask=masks[g])
            plsc.store_scatter(l_buf, [pis[g] + R], ams[g], mask=masks[g])

        out_copies.extend([
            pltpu.async_copy(
                l_buf, l_hbm.at[pl.ds((b0 + j) * 2 * R, 2 * R)], sem_out),
            pltpu.async_copy(
                b_buf, b_hbm.at[pl.ds((b0 + j) * 2 * R * 4, 2 * R * 4)],
                sem_out),
        ])

    out_copies.append(pltpu.async_copy(
        scores_v, sc_hbm.at[pl.ds(b0 * R, BPW * R)], sem_out))
    for c in out_copies:
        c.wait()


def _vs_body(act_ref, sc_ref, vs_ref):
    vs_ref[...] = jax.nn.sigmoid(act_ref[...]) * sc_ref[...][:, :, None]


_vs_tc = pl.pallas_call(
    _vs_body,
    out_shape=jax.ShapeDtypeStruct((B, R, V), jnp.float32),
    grid=(B // 8,),
    in_specs=[
        pl.BlockSpec((8, R, V), lambda i: (i, 0, 0)),
        pl.BlockSpec((8, R), lambda i: (i, 0)),
    ],
    out_specs=pl.BlockSpec((8, R, V), lambda i: (i, 0, 0)),
)


def kernel(pred_logits, pred_boxes, pred_actions, pred_rel_pairs, target_sizes):
    l1, b1, sc1 = _scpart(
        pred_logits, pred_boxes.reshape(-1),
        pred_rel_pairs.reshape(-1), target_sizes.reshape(-1))
    vs = _vs_tc(pred_actions, sc1.reshape(B, R))
    return (l1.reshape(B, 2 * R), b1.reshape(B, 2 * R, 4), vs)


# speedup vs baseline: 1.3396x; 1.2112x over previous
import functools

import jax
import jax.numpy as jnp
from jax import lax
from jax.experimental import pallas as pl
from jax.experimental.pallas import tpu as pltpu
from jax.experimental.pallas import tpu_sc as plsc

B, Q, C, R, V = 64, 300, 92, 100, 117
NC, NS, L = 2, 16, 16
NW = NC * NS
BPW = B // NW
NG = (R + L - 1) // L
NCLS = 81

_mesh = plsc.VectorSubcoreMesh(
    core_axis_name="c", subcore_axis_name="s", num_cores=NC, num_subcores=NS)


def _splat(x, dtype=jnp.int32):
    return jnp.zeros((L,), dtype) + x


def _iota():
    return lax.iota(jnp.int32, L)


def _batch_scratch():
    return [
        pltpu.VMEM((R * 2,), jnp.int32),
        pltpu.VMEM((Q, C), jnp.float32),
        pltpu.VMEM((Q * 4,), jnp.float32),
        pltpu.VMEM((2 * R,), jnp.int32),
        pltpu.VMEM((2 * R * 4,), jnp.float32),
    ]


@functools.partial(
    pl.kernel,
    out_type=(
        jax.ShapeDtypeStruct((B * 2 * R,), jnp.int32),
        jax.ShapeDtypeStruct((B * 2 * R * 4,), jnp.float32),
        jax.ShapeDtypeStruct((B * R,), jnp.float32),
    ),
    mesh=_mesh,
    compiler_params=pltpu.CompilerParams(
        needs_layout_passes=False, use_tc_tiling_on_sc=True),
    scratch_types=[pltpu.VMEM((B * 2,), jnp.float32),
                   pltpu.VMEM((BPW * R,), jnp.float32)] + _batch_scratch()
    + _batch_scratch()
    + [pltpu.SemaphoreType.DMA, pltpu.SemaphoreType.DMA,
       pltpu.SemaphoreType.DMA],
)
def _scpart(logits_hbm, boxes_hbm, pairs_hbm, ts_hbm,
            l_hbm, b_hbm, sc_hbm, ts_v, scores_v, *rest):
    per_batch = (rest[0:5], rest[5:10])
    sems = rest[10:12]
    sem_out = rest[12]
    wid = lax.axis_index("s") * NC + lax.axis_index("c")
    b0 = wid * BPW

    copies = []
    for j in range(BPW):
        pairs_v, logits_v, boxes_v = per_batch[j][:3]
        bj = b0 + j
        copies.append([
            pltpu.async_copy(logits_hbm.at[bj], logits_v, sems[j]),
            pltpu.async_copy(
                boxes_hbm.at[pl.ds(bj * Q * 4, Q * 4)], boxes_v, sems[j]),
            pltpu.async_copy(
                pairs_hbm.at[pl.ds(bj * R * 2, R * 2)], pairs_v, sems[j]),
        ])
    pltpu.sync_copy(ts_hbm, ts_v)
    out_copies = []

    for j in range(BPW):
        pairs_v, logits_v, boxes_v, l_buf, b_buf = per_batch[j]
        for c in copies[j]:
            c.wait()
        img_h = plsc.load_gather(ts_v, [_splat((b0 + j) * 2 + 0)])
        img_w = plsc.load_gather(ts_v, [_splat((b0 + j) * 2 + 1)])

        pis, masks, ois = [], [], []
        for g in range(NG):
            pi_raw = _iota() + g * L
            masks.append((pi_raw < R) if g == NG - 1 else None)
            pis.append(jnp.minimum(pi_raw, R - 1))
            ois.append(plsc.load_gather(pairs_v, [pis[g] * 2 + 1]))

        @plsc.parallel_loop(0, NG)
        def _boxes(g, pairs_v=pairs_v, boxes_v=boxes_v, l_buf=l_buf,
                   b_buf=b_buf, img_h=img_h, img_w=img_w):
            pi_raw = _iota() + g * L
            mask = pi_raw < R
            pi = jnp.minimum(pi_raw, R - 1)
            plsc.store_scatter(l_buf, [pi], _splat(0), mask=mask)
            for side in range(2):
                bi = plsc.load_gather(pairs_v, [pi * 2 + side])
                cx = plsc.load_gather(boxes_v, [bi * 4 + 0])
                cy = plsc.load_gather(boxes_v, [bi * 4 + 1])
                w = plsc.load_gather(boxes_v, [bi * 4 + 2])
                h = plsc.load_gather(boxes_v, [bi * 4 + 3])
                ob = pi + side * R
                plsc.store_scatter(b_buf, [ob + 0 * 2 * R],
                                   (cx - 0.5 * w) * img_w, mask=mask)
                plsc.store_scatter(b_buf, [ob + 1 * 2 * R],
                                   (cy - 0.5 * h) * img_h, mask=mask)
                plsc.store_scatter(b_buf, [ob + 2 * 2 * R],
                                   (cx + 0.5 * w) * img_w, mask=mask)
                plsc.store_scatter(b_buf, [ob + 3 * 2 * R],
                                   (cy + 0.5 * h) * img_h, mask=mask)

        init = (tuple(_splat(-jnp.inf, jnp.float32) for _ in range(NG)),
                tuple(_splat(0) for _ in range(NG)))

        @plsc.parallel_loop(0, NCLS - 1, unroll=4, carry=init)
        def maxstate(c, carry, ois=ois, logits_v=logits_v):
            ms, ams = carry
            ms, ams = list(ms), list(ams)
            cs = _splat(c)
            for g in range(NG):
                v = plsc.load_gather(logits_v, [ois[g], cs])
                gt = v > ms[g]
                ms[g] = jnp.where(gt, v, ms[g])
                ams[g] = jnp.where(gt, cs, ams[g])
            return tuple(ms), tuple(ams)

        m80s, ams = maxstate
        malls = []
        for g in range(NG):
            v80 = plsc.load_gather(logits_v, [ois[g], _splat(NCLS - 1)])
            malls.append(jnp.maximum(m80s[g], v80))
        malls = tuple(malls)

        @plsc.parallel_loop(0, NCLS, unroll=3,
                            carry=tuple(_splat(0.0, jnp.float32)
                                        for _ in range(NG)))
        def ss(c, ss_c, ois=ois, malls=malls, logits_v=logits_v):
            ss_c = list(ss_c)
            cs = _splat(c)
            for g in range(NG):
                v = plsc.load_gather(logits_v, [ois[g], cs])
                ss_c[g] = ss_c[g] + jnp.exp(v - malls[g])
            return tuple(ss_c)

        for g in range(NG):
            score = jnp.exp(m80s[g] - malls[g]) / ss[g]
            plsc.store_scatter(scores_v, [pis[g] + j * R], score,
                               mask=masks[g])
            plsc.store_scatter(l_buf, [pis[g] + R], ams[g], mask=masks[g])

        out_copies.extend([
            pltpu.async_copy(
                l_buf, l_hbm.at[pl.ds((b0 + j) * 2 * R, 2 * R)], sem_out),
            pltpu.async_copy(
                b_buf, b_hbm.at[pl.ds((b0 + j) * 2 * R * 4, 2 * R * 4)],
                sem_out),
        ])

    out_copies.append(pltpu.async_copy(
        scores_v, sc_hbm.at[pl.ds(b0 * R, BPW * R)], sem_out))
    for c in out_copies:
        c.wait()


def _vs_body(act_ref, sc_ref, vs_ref):
    i = pl.program_id(0)
    sc = sc_ref[pl.ds(i * 20, 20), :]
    vs_ref[...] = jax.nn.sigmoid(act_ref[...]) * sc[:, :, None]


_vs_tc = pl.pallas_call(
    _vs_body,
    out_shape=jax.ShapeDtypeStruct((R, B, V), jnp.float32),
    grid=(R // 20,),
    in_specs=[
        pl.BlockSpec((20, B, V), lambda i: (i, 0, 0)),
        pl.BlockSpec((R, B), lambda i: (0, 0)),
    ],
    out_specs=pl.BlockSpec((20, B, V), lambda i: (i, 0, 0)),
)


def kernel(pred_logits, pred_boxes, pred_actions, pred_rel_pairs, target_sizes):
    l1, b1, sc1 = _scpart(
        pred_logits, pred_boxes.reshape(-1),
        pred_rel_pairs.reshape(-1), target_sizes.reshape(-1))
    vs_t = _vs_tc(pred_actions.transpose(1, 0, 2), sc1.reshape(B, R).T)
    return (l1.reshape(B, 2 * R),
            b1.reshape(B, 4, 2 * R).transpose(0, 2, 1),
            vs_t.transpose(1, 0, 2))


# speedup vs baseline: 1.8779x; 1.4018x over previous
import functools

import jax
import jax.numpy as jnp
from jax import lax
from jax.experimental import pallas as pl
from jax.experimental.pallas import tpu as pltpu
from jax.experimental.pallas import tpu_sc as plsc

B, Q, C, R, V = 64, 300, 92, 100, 117
NC, NS, L = 2, 16, 16
NW = NC * NS
BPW = B // NW
NG = (R + L - 1) // L
NCLS = 81

_mesh = plsc.VectorSubcoreMesh(
    core_axis_name="c", subcore_axis_name="s", num_cores=NC, num_subcores=NS)


def _splat(x, dtype=jnp.int32):
    return jnp.zeros((L,), dtype) + x


def _iota():
    return lax.iota(jnp.int32, L)


def _batch_scratch():
    return [
        pltpu.VMEM((R * 2,), jnp.int32),
        pltpu.VMEM((C, Q), jnp.float32),
        pltpu.VMEM((Q * 4,), jnp.float32),
        pltpu.VMEM((2 * R,), jnp.int32),
        pltpu.VMEM((2 * R * 4,), jnp.float32),
    ]


@functools.partial(
    pl.kernel,
    out_type=(
        jax.ShapeDtypeStruct((B * 2 * R,), jnp.int32),
        jax.ShapeDtypeStruct((B * 2 * R * 4,), jnp.float32),
        jax.ShapeDtypeStruct((B * R,), jnp.float32),
    ),
    mesh=_mesh,
    compiler_params=pltpu.CompilerParams(
        needs_layout_passes=False, use_tc_tiling_on_sc=True),
    scratch_types=[pltpu.VMEM((B * 2,), jnp.float32),
                   pltpu.VMEM((BPW * R,), jnp.float32)] + _batch_scratch()
    + _batch_scratch()
    + [pltpu.SemaphoreType.DMA, pltpu.SemaphoreType.DMA,
       pltpu.SemaphoreType.DMA],
)
def _scpart(logits_hbm, boxes_hbm, pairs_hbm, ts_hbm,
            l_hbm, b_hbm, sc_hbm, ts_v, scores_v, *rest):
    per_batch = (rest[0:5], rest[5:10])
    sems = rest[10:12]
    sem_out = rest[12]
    wid = lax.axis_index("s") * NC + lax.axis_index("c")
    b0 = wid * BPW

    copies = []
    for j in range(BPW):
        pairs_v, logits_v, boxes_v = per_batch[j][:3]
        bj = b0 + j
        copies.append([
            pltpu.async_copy(logits_hbm.at[:, bj], logits_v, sems[j]),
            pltpu.async_copy(
                boxes_hbm.at[pl.ds(bj * Q * 4, Q * 4)], boxes_v, sems[j]),
            pltpu.async_copy(
                pairs_hbm.at[pl.ds(bj * R * 2, R * 2)], pairs_v, sems[j]),
        ])
    pltpu.sync_copy(ts_hbm, ts_v)
    out_copies = []

    for j in range(BPW):
        pairs_v, logits_v, boxes_v, l_buf, b_buf = per_batch[j]
        for c in copies[j]:
            c.wait()
        img_h = plsc.load_gather(ts_v, [_splat((b0 + j) * 2 + 0)])
        img_w = plsc.load_gather(ts_v, [_splat((b0 + j) * 2 + 1)])

        pis, masks, ois = [], [], []
        for g in range(NG):
            pi_raw = _iota() + g * L
            masks.append((pi_raw < R) if g == NG - 1 else None)
            pis.append(jnp.minimum(pi_raw, R - 1))
            ois.append(plsc.load_gather(pairs_v, [pis[g] * 2 + 1]))

        @plsc.parallel_loop(0, NG)
        def _boxes(g, pairs_v=pairs_v, boxes_v=boxes_v, l_buf=l_buf,
                   b_buf=b_buf, img_h=img_h, img_w=img_w):
            pi_raw = _iota() + g * L
            mask = pi_raw < R
            pi = jnp.minimum(pi_raw, R - 1)
            plsc.store_scatter(l_buf, [pi], _splat(0), mask=mask)
            for side in range(2):
                bi = plsc.load_gather(pairs_v, [pi * 2 + side])
                cx = plsc.load_gather(boxes_v, [bi * 4 + 0])
                cy = plsc.load_gather(boxes_v, [bi * 4 + 1])
                w = plsc.load_gather(boxes_v, [bi * 4 + 2])
                h = plsc.load_gather(boxes_v, [bi * 4 + 3])
                ob = pi + side * R
                plsc.store_scatter(b_buf, [ob + 0 * 2 * R],
                                   (cx - 0.5 * w) * img_w, mask=mask)
                plsc.store_scatter(b_buf, [ob + 1 * 2 * R],
                                   (cy - 0.5 * h) * img_h, mask=mask)
                plsc.store_scatter(b_buf, [ob + 2 * 2 * R],
                                   (cx + 0.5 * w) * img_w, mask=mask)
                plsc.store_scatter(b_buf, [ob + 3 * 2 * R],
                                   (cy + 0.5 * h) * img_h, mask=mask)

        init = (tuple(_splat(-jnp.inf, jnp.float32) for _ in range(NG)),
                tuple(_splat(0) for _ in range(NG)))

        @plsc.parallel_loop(0, NCLS - 1, unroll=4, carry=init)
        def maxstate(c, carry, ois=ois, logits_v=logits_v):
            ms, ams = carry
            ms, ams = list(ms), list(ams)
            cs = _splat(c)
            for g in range(NG):
                v = plsc.load_gather(logits_v, [cs, ois[g]])
                gt = v > ms[g]
                ms[g] = jnp.where(gt, v, ms[g])
                ams[g] = jnp.where(gt, cs, ams[g])
            return tuple(ms), tuple(ams)

        m80s, ams = maxstate
        malls = []
        for g in range(NG):
            v80 = plsc.load_gather(logits_v, [_splat(NCLS - 1), ois[g]])
            malls.append(jnp.maximum(m80s[g], v80))
        malls = tuple(malls)

        @plsc.parallel_loop(0, NCLS, unroll=3,
                            carry=tuple(_splat(0.0, jnp.float32)
                                        for _ in range(NG)))
        def ss(c, ss_c, ois=ois, malls=malls, logits_v=logits_v):
            ss_c = list(ss_c)
            cs = _splat(c)
            for g in range(NG):
                v = plsc.load_gather(logits_v, [cs, ois[g]])
                ss_c[g] = ss_c[g] + jnp.exp(v - malls[g])
            return tuple(ss_c)

        for g in range(NG):
            score = jnp.exp(m80s[g] - malls[g]) / ss[g]
            plsc.store_scatter(scores_v, [pis[g] + j * R], score,
                               mask=masks[g])
            plsc.store_scatter(l_buf, [pis[g] + R], ams[g], mask=masks[g])

        out_copies.extend([
            pltpu.async_copy(
                l_buf, l_hbm.at[pl.ds((b0 + j) * 2 * R, 2 * R)], sem_out),
            pltpu.async_copy(
                b_buf, b_hbm.at[pl.ds((b0 + j) * 2 * R * 4, 2 * R * 4)],
                sem_out),
        ])

    out_copies.append(pltpu.async_copy(
        scores_v, sc_hbm.at[pl.ds(b0 * R, BPW * R)], sem_out))
    for c in out_copies:
        c.wait()


def _vs_body(act_ref, sc_ref, vs_ref):
    i = pl.program_id(0)
    sc = sc_ref[pl.ds(i * 20, 20), :]
    vs_ref[...] = jax.nn.sigmoid(act_ref[...]) * sc[:, :, None]


_vs_tc = pl.pallas_call(
    _vs_body,
    out_shape=jax.ShapeDtypeStruct((R, B, V), jnp.float32),
    grid=(R // 20,),
    in_specs=[
        pl.BlockSpec((20, B, V), lambda i: (i, 0, 0)),
        pl.BlockSpec((R, B), lambda i: (0, 0)),
    ],
    out_specs=pl.BlockSpec((20, B, V), lambda i: (i, 0, 0)),
)


def kernel(pred_logits, pred_boxes, pred_actions, pred_rel_pairs, target_sizes):
    l1, b1, sc1 = _scpart(
        pred_logits.transpose(2, 0, 1), pred_boxes.reshape(-1),
        pred_rel_pairs.reshape(-1), target_sizes.reshape(-1))
    vs_t = _vs_tc(pred_actions.transpose(1, 0, 2), sc1.reshape(B, R).T)
    return (l1.reshape(B, 2 * R),
            b1.reshape(B, 4, 2 * R).transpose(0, 2, 1),
            vs_t.transpose(1, 0, 2))
